# baseline (speedup 1.0000x reference)
import functools

import jax
import jax.numpy as jnp
from jax import lax
from jax.experimental import pallas as pl
from jax.experimental.pallas import tpu as pltpu
from jax.experimental.pallas import tpu_sc as plsc

N = 10000
E = 320000
DIN = 128
DH = DIN // 2
DE = 16
G = 64
L = 4

NC = 2
NS = 16
NW = NC * NS
EPW = E // NW
CHUNK = 80
NCHUNK = EPW // CHUNK
ROWS_PW = 624
ROWS_TAIL = N - NS * ROWS_PW
ZROWS = 48


_EBLK = 2000


def _edge_mm_body(ea_ref, w_ref, b_ref, out_ref):
    res = (jnp.dot(ea_ref[...], w_ref[...], preferred_element_type=jnp.float32)
           + b_ref[...])
    out_ref[0] = res[:, :DH]
    out_ref[1] = res[:, DH:]


def _edge_transform_one(edge_attr, w, b):
    return pl.pallas_call(
        _edge_mm_body,
        grid=(E // _EBLK,),
        in_specs=[
            pl.BlockSpec((_EBLK, DE), lambda i: (i, 0)),
            pl.BlockSpec((DE, DIN), lambda i: (0, 0)),
            pl.BlockSpec((1, DIN), lambda i: (0, 0)),
        ],
        out_specs=pl.BlockSpec((2, _EBLK, DH), lambda i: (0, i, 0)),
        out_shape=jax.ShapeDtypeStruct((2, E, DH), jnp.float32),
    )(edge_attr, w, b.reshape(1, DIN))


_sc_mesh = plsc.VectorSubcoreMesh(
    core_axis_name="c", subcore_axis_name="s", num_cores=NC, num_subcores=NS)


@functools.partial(
    pl.kernel,
    out_type=jax.ShapeDtypeStruct((NC, 2, N, DH), jnp.float32),
    mesh=_sc_mesh,
    scratch_types=[
        pltpu.VMEM_SHARED((N, DH), jnp.float32),
        pltpu.VMEM_SHARED((N, DH), jnp.float32),
        pltpu.VMEM((CHUNK,), jnp.int32),
        pltpu.VMEM((CHUNK,), jnp.int32),
        pltpu.VMEM((CHUNK,), jnp.int32),
        pltpu.VMEM((CHUNK,), jnp.int32),
        pltpu.VMEM((CHUNK,), jnp.int32),
        pltpu.VMEM((CHUNK,), jnp.int32),
        pltpu.VMEM((CHUNK,), jnp.int32),
        pltpu.VMEM((CHUNK,), jnp.int32),
        pltpu.VMEM((CHUNK, DH), jnp.float32),
        pltpu.VMEM((CHUNK, DH), jnp.float32),
        pltpu.VMEM((CHUNK, DH), jnp.float32),
        pltpu.VMEM((CHUNK, DH), jnp.float32),
        pltpu.VMEM((ZROWS, DH), jnp.float32),
        pltpu.SemaphoreType.DMA,
        pltpu.SemaphoreType.DMA,
        pltpu.SemaphoreType.DMA,
        pltpu.SemaphoreType.DMA,
        pltpu.SemaphoreType.DMA,
        pltpu.SemaphoreType.DMA,
        pltpu.SemaphoreType.DMA,
        pltpu.SemaphoreType.DMA,
        pltpu.SemaphoreType.DMA,
        pltpu.SemaphoreType.DMA,
        pltpu.SemaphoreType.DMA,
        pltpu.SemaphoreType.DMA,
        pltpu.SemaphoreType.DMA,
        pltpu.SemaphoreType.DMA,
        pltpu.SemaphoreType.DMA,
    ],
)
def _sc_message_pass(h2_hbm, src_hbm, dst_hbm, e2_hbm, out_hbm,
                     h_sh, aggr, src0, src1, src2, src3,
                     dst0, dst1, dst2, dst3,
                     e0, e1, g0, g1, z_v,
                     ss0, ss1, ss2, ss3, sd0, sd1, sd2, sd3,
                     se0, se1, sg0, sg1, sc0, sc1, sh):
    cid = lax.axis_index("c")
    sid = lax.axis_index("s")
    wid = sid * NC + cid

    src_q = (src0, src1, src2, src3)
    dst_q = (dst0, dst1, dst2, dst3)
    e_b = (e0, e1)
    g_b = (g0, g1)
    ss_q = (ss0, ss1, ss2, ss3)
    sd_q = (sd0, sd1, sd2, sd3)
    se_b = (se0, se1)
    sg_b = (sg0, sg1)
    sc_b = (sc0, sc1)

    zero = jnp.zeros((16,), jnp.float32)
    def _zrow(r, _):
        for j in range(DH // 16):
            z_v[r, pl.ds(j * 16, 16)] = zero
        return 0
    lax.fori_loop(0, ZROWS, _zrow, 0)

    base0 = wid * EPW
    rbase = sid * ROWS_PW

    def _start_idx(chunk, q):
        base = base0 + chunk * CHUNK
        pltpu.async_copy(src_hbm.at[pl.ds(base, CHUNK)], src_q[q], ss_q[q])
        pltpu.async_copy(dst_hbm.at[pl.ds(base, CHUNK)], dst_q[q], sd_q[q])

    def _wait_idx(q):
        pltpu.make_async_copy(src_hbm.at[pl.ds(0, CHUNK)], src_q[q], ss_q[q]).wait()
        pltpu.make_async_copy(dst_hbm.at[pl.ds(0, CHUNK)], dst_q[q], sd_q[q]).wait()

    def _compute(b):
        e_v, g_v = e_b[b], g_b[b]
        def _rows(r4, _):
            for dr in range(4):
                r = r4 * 4 + dr
                for j in range(DH // 16):
                    sl = pl.ds(j * 16, 16)
                    e_v[r, sl] = jnp.maximum(e_v[r, sl] + g_v[r, sl], 0.0)
            return 0
        lax.fori_loop(0, CHUNK // 4, _rows, 0)

    for p in range(2):
        pltpu.async_copy(h2_hbm.at[p, pl.ds(rbase, ROWS_PW)],
                         h_sh.at[pl.ds(rbase, ROWS_PW)], sh)
        def _zcopy(k, _):
            pltpu.sync_copy(z_v, aggr.at[pl.ds(rbase + k * ZROWS, ZROWS)])
            return 0
        lax.fori_loop(0, ROWS_PW // ZROWS, _zcopy, 0)

        @pl.when(sid == 0)
        def _zero_tail():
            pltpu.sync_copy(z_v.at[pl.ds(0, ROWS_TAIL)],
                            aggr.at[pl.ds(NS * ROWS_PW, ROWS_TAIL)])
            pltpu.sync_copy(h2_hbm.at[p, pl.ds(NS * ROWS_PW, ROWS_TAIL)],
                            h_sh.at[pl.ds(NS * ROWS_PW, ROWS_TAIL)])

        pltpu.make_async_copy(h2_hbm.at[p, pl.ds(rbase, ROWS_PW)],
                              h_sh.at[pl.ds(rbase, ROWS_PW)], sh).wait()
        plsc.subcore_barrier()

        def _start_eg(chunk, b, q):
            base = base0 + chunk * CHUNK
            pltpu.async_copy(e2_hbm.at[p, pl.ds(base, CHUNK)], e_b[b], se_b[b])
            pltpu.async_copy(h_sh.at[src_q[q]], g_b[b], sg_b[b])

        def _wait_eg(b, q):
            pltpu.make_async_copy(e2_hbm.at[p, pl.ds(0, CHUNK)], e_b[b], se_b[b]).wait()
            pltpu.make_async_copy(h_sh.at[src_q[q]], g_b[b], sg_b[b]).wait()

        def _wait_scatter(b, q):
            pltpu.make_async_copy(e_b[b], aggr.at[dst_q[q]], sc_b[b]).wait()

        _start_idx(0, 0)
        _start_idx(1, 1)
        _start_idx(2, 2)
        _wait_idx(0)
        _start_eg(0, 0, 0)

        def _quad(io, _):
            for k in range(4):
                i = io * 4 + k
                b = k % 2
                nb = 1 - b
                if k == 0:
                    @pl.when(io > 0)
                    def _():
                        _wait_scatter(nb, 3)
                else:
                    _wait_scatter(nb, (k + 3) % 4)
                _wait_idx((k + 1) % 4)
                _start_eg(i + 1, nb, (k + 1) % 4)
                @pl.when(i + 3 < NCHUNK)
                def _():
                    _start_idx(i + 3, (k + 3) % 4)
                _wait_eg(b, k)
                _compute(b)
                pltpu.async_copy(e_b[b], aggr.at[dst_q[k]], sc_b[b], add=True)
            return 0

        lax.fori_loop(0, (NCHUNK - 1) // 4, _quad, 0)

        _wait_scatter(1, 3)
        _wait_eg(0, 0)
        _compute(0)
        pltpu.sync_copy(e_b[0], aggr.at[dst_q[0]], add=True)
        plsc.subcore_barrier()

        def _ocopy(k, _):
            off = rbase + k * ZROWS
            pltpu.sync_copy(aggr.at[pl.ds(off, ZROWS)], z_v)
            pltpu.sync_copy(z_v, out_hbm.at[cid, p, pl.ds(off, ZROWS)])
            return 0
        lax.fori_loop(0, ROWS_PW // ZROWS, _ocopy, 0)

        @pl.when(sid == 0)
        def _copy_tail():
            pltpu.sync_copy(aggr.at[pl.ds(NS * ROWS_PW, ROWS_TAIL)],
                            z_v.at[pl.ds(0, ROWS_TAIL)])
            pltpu.sync_copy(z_v.at[pl.ds(0, ROWS_TAIL)],
                            out_hbm.at[cid, p, pl.ds(NS * ROWS_PW, ROWS_TAIL)])
            def _rz(r, _):
                for j in range(DH // 16):
                    z_v[r, pl.ds(j * 16, 16)] = zero
                return 0
            lax.fori_loop(0, ROWS_TAIL, _rz, 0)

        plsc.subcore_barrier()


def _layer_dense_body(h2_ref, part_ref, eps_ref, w1_ref, b1_ref, w2_ref,
                      b2_ref, g_ref, bb_ref, out_ref):
    h = jnp.concatenate([h2_ref[0], h2_ref[1]], axis=1)
    aggr = jnp.concatenate([part_ref[0, 0] + part_ref[1, 0],
                            part_ref[0, 1] + part_ref[1, 1]], axis=1)
    out = (1.0 + eps_ref[0, 0]) * h + aggr
    t = jnp.maximum(jnp.dot(out, w1_ref[...], preferred_element_type=jnp.float32)
                    + b1_ref[...], 0.0)
    t = jnp.dot(t, w2_ref[...], preferred_element_type=jnp.float32) + b2_ref[...]
    mean = jnp.mean(t, axis=0, keepdims=True)
    var = jnp.mean(t * t, axis=0, keepdims=True) - mean * mean
    t = (t - mean) * jax.lax.rsqrt(var + 1e-5) * g_ref[...] + bb_ref[...]
    y = t * jax.nn.sigmoid(t)
    out_ref[0] = y[:, :DH]
    out_ref[1] = y[:, DH:]


def _layer_dense(h2, part, eps_l, w1, b1, w2, b2, bn_g, bn_b):
    return pl.pallas_call(
        _layer_dense_body,
        out_shape=jax.ShapeDtypeStruct((2, N, DH), jnp.float32),
    )(h2, part, eps_l.reshape(1, 1), w1, b1.reshape(1, DIN), w2,
      b2.reshape(1, DIN), bn_g.reshape(1, DIN), bn_b.reshape(1, DIN))


def _head_body(h2_ref, batch_ref, fc1w_ref, fc1b_ref, fc2w_ref, fc2b_ref, out_ref):
    h = jnp.concatenate([h2_ref[0], h2_ref[1]], axis=1)
    b = batch_ref[...]
    seg = lax.broadcasted_iota(jnp.int32, (G, N), 0)
    onehot = (seg == b).astype(jnp.float32)
    sums = jnp.dot(onehot, h, preferred_element_type=jnp.float32)
    counts = jnp.sum(onehot, axis=1, keepdims=True)
    pooled = sums / jnp.maximum(counts, 1.0)
    z = jnp.dot(pooled, fc1w_ref[...], preferred_element_type=jnp.float32) + fc1b_ref[...]
    z = z * jax.nn.sigmoid(z)
    z = jnp.dot(z, fc2w_ref[...], preferred_element_type=jnp.float32) + fc2b_ref[...]
    out_ref[...] = jax.nn.sigmoid(z)


def _head(h2, batch, fc1_W, fc1_b, fc2_W, fc2_b):
    return pl.pallas_call(
        _head_body,
        out_shape=jax.ShapeDtypeStruct((G, 1), jnp.float32),
    )(h2, batch.reshape(1, N), fc1_W, fc1_b.reshape(1, 64), fc2_W,
      fc2_b.reshape(1, 1))


def kernel(x, edge_index, edge_attr, batch, eps, We, be, W1, b1, W2, b2,
           bn_g, bn_b, fc1_W, fc1_b, fc2_W, fc2_b):
    e_list = [_edge_transform_one(edge_attr, We[l], be[l]) for l in range(L)]
    src = edge_index[0]
    dst = edge_index[1]
    h2 = jnp.stack([x[:, :DH], x[:, DH:]], axis=0)
    for l in range(L):
        part = _sc_message_pass(h2, src, dst, e_list[l])
        h2 = _layer_dense(h2, part, eps[l], W1[l], b1[l], W2[l], b2[l],
                          bn_g[l], bn_b[l])
    return _head(h2, batch, fc1_W, fc1_b, fc2_W, fc2_b)


# speedup vs baseline: 1.3677x; 1.3677x over previous
import functools

import jax
import jax.numpy as jnp
from jax import lax
from jax.experimental import pallas as pl
from jax.experimental.pallas import tpu as pltpu
from jax.experimental.pallas import tpu_sc as plsc

N = 10000
E = 320000
DIN = 128
DE = 16
G = 64
L = 4

NC = 2
NS = 16
NW = NC * NS
EPW = E // NW
CHUNK = 80
NCHUNK = EPW // CHUNK
ROWS_PW = 624
ROWS_TAIL = N - NS * ROWS_PW
ZROWS = 48

_PERM = ([32 * (w // 16) + (w % 16) for w in range(DIN // 2)]
         + [32 * (w // 16) + 16 + (w % 16) for w in range(DIN // 2)])


_EBLK = 2000


def _edge_mm_body(ea_ref, w_ref, b_ref, out_ref):
    res = (jnp.dot(ea_ref[...], w_ref[...], preferred_element_type=jnp.float32)
           + b_ref[...])
    lo = res[:, :DIN // 2].astype(jnp.bfloat16)
    hi = res[:, DIN // 2:].astype(jnp.bfloat16)
    ulo = lax.bitcast_convert_type(lo, jnp.uint16).astype(jnp.uint32)
    uhi = lax.bitcast_convert_type(hi, jnp.uint16).astype(jnp.uint32)
    out_ref[...] = lax.bitcast_convert_type(
        ulo | (uhi << jnp.uint32(16)), jnp.int32)


def _edge_transform_one(edge_attr, w, b):
    return pl.pallas_call(
        _edge_mm_body,
        grid=(E // _EBLK,),
        in_specs=[
            pl.BlockSpec((_EBLK, DE), lambda i: (i, 0)),
            pl.BlockSpec((DE, DIN), lambda i: (0, 0)),
            pl.BlockSpec((1, DIN), lambda i: (0, 0)),
        ],
        out_specs=pl.BlockSpec((_EBLK, DIN // 2), lambda i: (i, 0)),
        out_shape=jax.ShapeDtypeStruct((E, DIN // 2), jnp.int32),
    )(edge_attr, w, b.reshape(1, DIN))


_sc_mesh = plsc.VectorSubcoreMesh(
    core_axis_name="c", subcore_axis_name="s", num_cores=NC, num_subcores=NS)


@functools.partial(
    pl.kernel,
    out_type=jax.ShapeDtypeStruct((NC, N, DIN), jnp.float32),
    mesh=_sc_mesh,
    scratch_types=[
        pltpu.VMEM_SHARED((N, DIN), jnp.float32),
        pltpu.VMEM((CHUNK,), jnp.int32),
        pltpu.VMEM((CHUNK,), jnp.int32),
        pltpu.VMEM((CHUNK,), jnp.int32),
        pltpu.VMEM((CHUNK,), jnp.int32),
        pltpu.VMEM((CHUNK,), jnp.int32),
        pltpu.VMEM((CHUNK,), jnp.int32),
        pltpu.VMEM((CHUNK,), jnp.int32),
        pltpu.VMEM((CHUNK,), jnp.int32),
        pltpu.VMEM((CHUNK, DIN // 2), jnp.int32),
        pltpu.VMEM((CHUNK, DIN // 2), jnp.int32),
        pltpu.VMEM((CHUNK, DIN), jnp.float32),
        pltpu.VMEM((CHUNK, DIN), jnp.float32),
        pltpu.VMEM((ZROWS, DIN), jnp.float32),
        pltpu.SemaphoreType.DMA,
        pltpu.SemaphoreType.DMA,
        pltpu.SemaphoreType.DMA,
        pltpu.SemaphoreType.DMA,
        pltpu.SemaphoreType.DMA,
        pltpu.SemaphoreType.DMA,
        pltpu.SemaphoreType.DMA,
        pltpu.SemaphoreType.DMA,
        pltpu.SemaphoreType.DMA,
        pltpu.SemaphoreType.DMA,
        pltpu.SemaphoreType.DMA,
        pltpu.SemaphoreType.DMA,
        pltpu.SemaphoreType.DMA,
        pltpu.SemaphoreType.DMA,
    ],
)
def _sc_message_pass(h_hbm, src_hbm, dst_hbm, e_hbm, out_hbm,
                     aggr, src0, src1, src2, src3, dst0, dst1, dst2, dst3,
                     e0, e1, g0, g1, z_v,
                     ss0, ss1, ss2, ss3, sd0, sd1, sd2, sd3,
                     se0, se1, sg0, sg1, sc0, sc1):
    cid = lax.axis_index("c")
    sid = lax.axis_index("s")
    wid = sid * NC + cid

    src_q = (src0, src1, src2, src3)
    dst_q = (dst0, dst1, dst2, dst3)
    e_b = (e0, e1)
    g_b = (g0, g1)
    ss_q = (ss0, ss1, ss2, ss3)
    sd_q = (sd0, sd1, sd2, sd3)
    se_b = (se0, se1)
    sg_b = (sg0, sg1)
    sc_b = (sc0, sc1)

    zero = jnp.zeros((16,), jnp.float32)
    def _zrow(r, _):
        for j in range(DIN // 16):
            z_v[r, pl.ds(j * 16, 16)] = zero
        return 0
    lax.fori_loop(0, ZROWS, _zrow, 0)
    def _zcopy(k, _):
        pltpu.sync_copy(z_v, aggr.at[pl.ds(sid * ROWS_PW + k * ZROWS, ZROWS)])
        return 0
    lax.fori_loop(0, ROWS_PW // ZROWS, _zcopy, 0)

    @pl.when(sid == 0)
    def _zero_tail():
        pltpu.sync_copy(z_v.at[pl.ds(0, ROWS_TAIL)],
                        aggr.at[pl.ds(NS * ROWS_PW, ROWS_TAIL)])

    plsc.subcore_barrier()

    base0 = wid * EPW

    def _start_idx(chunk, q):
        base = base0 + chunk * CHUNK
        pltpu.async_copy(src_hbm.at[pl.ds(base, CHUNK)], src_q[q], ss_q[q])
        pltpu.async_copy(dst_hbm.at[pl.ds(base, CHUNK)], dst_q[q], sd_q[q])

    def _start_eg(chunk, b, q):
        base = base0 + chunk * CHUNK
        pltpu.async_copy(e_hbm.at[pl.ds(base, CHUNK)], e_b[b], se_b[b])
        pltpu.async_copy(h_hbm.at[src_q[q]], g_b[b], sg_b[b])

    def _wait_idx(q):
        pltpu.make_async_copy(src_hbm.at[pl.ds(0, CHUNK)], src_q[q], ss_q[q]).wait()
        pltpu.make_async_copy(dst_hbm.at[pl.ds(0, CHUNK)], dst_q[q], sd_q[q]).wait()

    def _wait_eg(b, q):
        pltpu.make_async_copy(e_hbm.at[pl.ds(0, CHUNK)], e_b[b], se_b[b]).wait()
        pltpu.make_async_copy(h_hbm.at[src_q[q]], g_b[b], sg_b[b]).wait()

    def _wait_scatter(b, q):
        pltpu.make_async_copy(g_b[b], aggr.at[dst_q[q]], sc_b[b]).wait()

    def _compute(b):
        e_v, g_v = e_b[b], g_b[b]
        def _rows(r4, _):
            for dr in range(4):
                r = r4 * 4 + dr
                for gg in range(DIN // 32):
                    v = e_v[r, pl.ds(gg * 16, 16)]
                    lo = lax.bitcast_convert_type(
                        jnp.left_shift(v, 16), jnp.float32)
                    hi = lax.bitcast_convert_type(
                        jnp.bitwise_and(v, jnp.int32(-65536)), jnp.float32)
                    sl0 = pl.ds(gg * 32, 16)
                    sl1 = pl.ds(gg * 32 + 16, 16)
                    g_v[r, sl0] = jnp.maximum(g_v[r, sl0] + lo, 0.0)
                    g_v[r, sl1] = jnp.maximum(g_v[r, sl1] + hi, 0.0)
            return 0
        lax.fori_loop(0, CHUNK // 4, _rows, 0)

    _start_idx(0, 0)
    _start_idx(1, 1)
    _start_idx(2, 2)
    _wait_idx(0)
    _start_eg(0, 0, 0)

    def _quad(io, _):
        for k in range(4):
            i = io * 4 + k
            b = k % 2
            nb = 1 - b
            if k == 0:
                @pl.when(io > 0)
                def _():
                    _wait_scatter(nb, 3)
            else:
                _wait_scatter(nb, (k + 3) % 4)
            _wait_idx((k + 1) % 4)
            _start_eg(i + 1, nb, (k + 1) % 4)
            @pl.when(i + 3 < NCHUNK)
            def _():
                _start_idx(i + 3, (k + 3) % 4)
            _wait_eg(b, k)
            _compute(b)
            pltpu.async_copy(g_b[b], aggr.at[dst_q[k]], sc_b[b], add=True)
        return 0

    lax.fori_loop(0, (NCHUNK - 1) // 4, _quad, 0)

    _wait_scatter(1, 3)
    _wait_eg(0, 0)
    _compute(0)
    pltpu.sync_copy(g_b[0], aggr.at[dst_q[0]], add=True)

    plsc.subcore_barrier()
    def _ocopy(k, _):
        off = sid * ROWS_PW + k * ZROWS
        pltpu.sync_copy(aggr.at[pl.ds(off, ZROWS)], z_v)
        pltpu.sync_copy(z_v, out_hbm.at[cid, pl.ds(off, ZROWS)])
        return 0
    lax.fori_loop(0, ROWS_PW // ZROWS, _ocopy, 0)

    @pl.when(sid == 0)
    def _copy_tail():
        pltpu.sync_copy(aggr.at[pl.ds(NS * ROWS_PW, ROWS_TAIL)],
                        z_v.at[pl.ds(0, ROWS_TAIL)])
        pltpu.sync_copy(z_v.at[pl.ds(0, ROWS_TAIL)],
                        out_hbm.at[cid, pl.ds(NS * ROWS_PW, ROWS_TAIL)])


def _layer_dense_body(h_ref, part_ref, eps_ref, w1_ref, b1_ref, w2_ref,
                      b2_ref, g_ref, bb_ref, out_ref):
    h = h_ref[...]
    out = (1.0 + eps_ref[0, 0]) * h + part_ref[0] + part_ref[1]
    t = jnp.maximum(jnp.dot(out, w1_ref[...], preferred_element_type=jnp.float32)
                    + b1_ref[...], 0.0)
    t = jnp.dot(t, w2_ref[...], preferred_element_type=jnp.float32) + b2_ref[...]
    mean = jnp.mean(t, axis=0, keepdims=True)
    var = jnp.mean(t * t, axis=0, keepdims=True) - mean * mean
    t = (t - mean) * jax.lax.rsqrt(var + 1e-5) * g_ref[...] + bb_ref[...]
    out_ref[...] = t * jax.nn.sigmoid(t)


def _layer_dense(h, part, eps_l, w1, b1, w2, b2, bn_g, bn_b):
    return pl.pallas_call(
        _layer_dense_body,
        out_shape=jax.ShapeDtypeStruct((N, DIN), jnp.float32),
    )(h, part, eps_l.reshape(1, 1), w1, b1.reshape(1, DIN), w2,
      b2.reshape(1, DIN), bn_g.reshape(1, DIN), bn_b.reshape(1, DIN))


def _head_body(h_ref, batch_ref, fc1w_ref, fc1b_ref, fc2w_ref, fc2b_ref, out_ref):
    b = batch_ref[...]
    seg = lax.broadcasted_iota(jnp.int32, (G, N), 0)
    onehot = (seg == b).astype(jnp.float32)
    sums = jnp.dot(onehot, h_ref[...], preferred_element_type=jnp.float32)
    counts = jnp.sum(onehot, axis=1, keepdims=True)
    pooled = sums / jnp.maximum(counts, 1.0)
    z = jnp.dot(pooled, fc1w_ref[...], preferred_element_type=jnp.float32) + fc1b_ref[...]
    z = z * jax.nn.sigmoid(z)
    z = jnp.dot(z, fc2w_ref[...], preferred_element_type=jnp.float32) + fc2b_ref[...]
    out_ref[...] = jax.nn.sigmoid(z)


def _head(h, batch, fc1_W, fc1_b, fc2_W, fc2_b):
    return pl.pallas_call(
        _head_body,
        out_shape=jax.ShapeDtypeStruct((G, 1), jnp.float32),
    )(h, batch.reshape(1, N), fc1_W, fc1_b.reshape(1, 64), fc2_W,
      fc2_b.reshape(1, 1))


def kernel(x, edge_index, edge_attr, batch, eps, We, be, W1, b1, W2, b2,
           bn_g, bn_b, fc1_W, fc1_b, fc2_W, fc2_b):
    perm = jnp.array(_PERM, dtype=jnp.int32)
    e_list = [_edge_transform_one(edge_attr, We[l][:, perm], be[l][perm])
              for l in range(L)]
    src = edge_index[0]
    dst = edge_index[1]
    h = x
    for l in range(L):
        part = _sc_message_pass(h, src, dst, e_list[l])
        h = _layer_dense(h, part, eps[l], W1[l], b1[l], W2[l], b2[l],
                         bn_g[l], bn_b[l])
    return _head(h, batch, fc1_W, fc1_b, fc2_W, fc2_b)
